# baseline (speedup 1.0000x reference)
import jax
import jax.numpy as jnp
from jax.experimental import pallas as pl
from jax.experimental.pallas import tpu as pltpu

_TILE_H = 8


def _head_kernel(x_ref, w_ref, b_ref, oc_ref, og_ref, od_ref, *, oc, og, od):
    b = b_ref[:]
    for h in range(x_ref.shape[2]):
        acc = jnp.dot(w_ref[:], x_ref[0, :, h, :],
                      preferred_element_type=jnp.float32) + b
        oc_ref[0, :, h, :] = acc[0:oc]
        og_ref[0, :, h, :] = acc[oc:oc + og]
        od_ref[0, :, h, :] = acc[oc + og:oc + og + od]


@jax.jit
def kernel(x, W_cls, b_cls, W_reg, b_reg, W_dir, b_dir):
    B, C, H, W = x.shape
    Oc = W_cls.shape[0]
    Og = W_reg.shape[0]
    Od = W_dir.shape[0]
    w_all = jnp.concatenate([W_cls, W_reg, W_dir], axis=0)
    b_all = jnp.concatenate([b_cls, b_reg, b_dir], axis=0).reshape(-1, 1)
    n_h = pl.cdiv(H, _TILE_H)

    def o_spec(o):
        return pl.BlockSpec((1, o, _TILE_H, W), lambda b, h: (b, 0, h, 0))

    import functools
    out_cls, out_reg, out_dir = pl.pallas_call(
        functools.partial(_head_kernel, oc=Oc, og=Og, od=Od),
        grid=(B, n_h),
        in_specs=[
            pl.BlockSpec((1, C, _TILE_H, W), lambda b, h: (b, 0, h, 0)),
            pl.BlockSpec((Oc + Og + Od, C), lambda b, h: (0, 0)),
            pl.BlockSpec((Oc + Og + Od, 1), lambda b, h: (0, 0)),
        ],
        out_specs=(o_spec(Oc), o_spec(Og), o_spec(Od)),
        out_shape=(
            jax.ShapeDtypeStruct((B, Oc, H, W), jnp.float32),
            jax.ShapeDtypeStruct((B, Og, H, W), jnp.float32),
            jax.ShapeDtypeStruct((B, Od, H, W), jnp.float32),
        ),
        compiler_params=pltpu.CompilerParams(
            dimension_semantics=("parallel", "parallel"),
        ),
    )(x, w_all, b_all)

    return (out_cls, out_reg, out_dir)


# speedup vs baseline: 1.2955x; 1.2955x over previous
import functools

import jax
import jax.numpy as jnp
from jax.experimental import pallas as pl
from jax.experimental.pallas import tpu as pltpu

_TILE_W = 8


def _head_kernel(x_ref, w_ref, b_ref, oc_ref, og_ref, od_ref, *, oc, og, od):
    b = b_ref[:]
    for w in range(x_ref.shape[2]):
        acc = jnp.dot(w_ref[:], x_ref[0, :, w, :],
                      preferred_element_type=jnp.float32) + b
        oc_ref[0, :, w, :] = acc[0:oc]
        og_ref[0, :, w, :] = acc[oc:oc + og]
        od_ref[0, :, w, :] = acc[oc + og:oc + og + od]


@jax.jit
def kernel(x, W_cls, b_cls, W_reg, b_reg, W_dir, b_dir):
    B, C, H, W = x.shape
    Oc = W_cls.shape[0]
    Og = W_reg.shape[0]
    Od = W_dir.shape[0]
    w_all = jnp.concatenate([W_cls, W_reg, W_dir], axis=0)
    b_all = jnp.concatenate([b_cls, b_reg, b_dir], axis=0).reshape(-1, 1)
    xt = jnp.swapaxes(x, 2, 3)
    n_w = pl.cdiv(W, _TILE_W)

    def o_spec(o):
        return pl.BlockSpec((1, o, _TILE_W, H), lambda b, w: (b, 0, w, 0))

    out_cls, out_reg, out_dir = pl.pallas_call(
        functools.partial(_head_kernel, oc=Oc, og=Og, od=Od),
        grid=(B, n_w),
        in_specs=[
            pl.BlockSpec((1, C, _TILE_W, H), lambda b, w: (b, 0, w, 0)),
            pl.BlockSpec((Oc + Og + Od, C), lambda b, w: (0, 0)),
            pl.BlockSpec((Oc + Og + Od, 1), lambda b, w: (0, 0)),
        ],
        out_specs=(o_spec(Oc), o_spec(Og), o_spec(Od)),
        out_shape=(
            jax.ShapeDtypeStruct((B, Oc, W, H), jnp.float32),
            jax.ShapeDtypeStruct((B, Og, W, H), jnp.float32),
            jax.ShapeDtypeStruct((B, Od, W, H), jnp.float32),
        ),
        compiler_params=pltpu.CompilerParams(
            dimension_semantics=("parallel", "parallel"),
        ),
    )(xt, w_all, b_all)

    return (jnp.swapaxes(out_cls, 2, 3),
            jnp.swapaxes(out_reg, 2, 3),
            jnp.swapaxes(out_dir, 2, 3))


# speedup vs baseline: 1.4127x; 1.0905x over previous
import jax
import jax.numpy as jnp
from jax.experimental import pallas as pl
from jax.experimental.pallas import tpu as pltpu

_TILE_S = 2048


def _head_kernel(x_ref, w_ref, b_ref, oc_ref, og_ref, od_ref):
    oc = oc_ref.shape[2]
    og = og_ref.shape[2]
    acc = jnp.dot(x_ref[0], w_ref[:],
                  preferred_element_type=jnp.float32) + b_ref[:]
    oc_ref[0] = acc[:, 0:oc]
    og_ref[0] = acc[:, oc:oc + og]
    od_ref[0] = acc[:, oc + og:]


@jax.jit
def kernel(x, W_cls, b_cls, W_reg, b_reg, W_dir, b_dir):
    B, C, H, W = x.shape
    HW = H * W
    Oc = W_cls.shape[0]
    Og = W_reg.shape[0]
    Od = W_dir.shape[0]
    O = Oc + Og + Od
    wt = jnp.concatenate([W_cls, W_reg, W_dir], axis=0).T
    b_all = jnp.concatenate([b_cls, b_reg, b_dir], axis=0).reshape(1, O)
    xv = jnp.transpose(x, (0, 2, 3, 1)).reshape(B, HW, C)
    n_s = pl.cdiv(HW, _TILE_S)

    def o_spec(o):
        return pl.BlockSpec((1, _TILE_S, o), lambda b, s: (b, s, 0))

    out_cls, out_reg, out_dir = pl.pallas_call(
        _head_kernel,
        grid=(B, n_s),
        in_specs=[
            pl.BlockSpec((1, _TILE_S, C), lambda b, s: (b, s, 0)),
            pl.BlockSpec((C, O), lambda b, s: (0, 0)),
            pl.BlockSpec((1, O), lambda b, s: (0, 0)),
        ],
        out_specs=(o_spec(Oc), o_spec(Og), o_spec(Od)),
        out_shape=(
            jax.ShapeDtypeStruct((B, HW, Oc), jnp.float32),
            jax.ShapeDtypeStruct((B, HW, Og), jnp.float32),
            jax.ShapeDtypeStruct((B, HW, Od), jnp.float32),
        ),
        compiler_params=pltpu.CompilerParams(
            dimension_semantics=("parallel", "parallel"),
        ),
    )(xv, wt, b_all)

    def to_nchw(o):
        return jnp.transpose(o.reshape(B, H, W, o.shape[2]), (0, 3, 1, 2))

    return (to_nchw(out_cls), to_nchw(out_reg), to_nchw(out_dir))


# speedup vs baseline: 1.9063x; 1.3494x over previous
import jax
import jax.numpy as jnp
from jax.experimental import pallas as pl
from jax.experimental.pallas import tpu as pltpu

_TILE_S = 2048


def _head_kernel(x_ref, w_ref, b_ref, out_ref):
    acc = jnp.dot(x_ref[0], w_ref[:],
                  preferred_element_type=jnp.float32) + b_ref[:]
    out_ref[0] = acc.T


@jax.jit
def kernel(x, W_cls, b_cls, W_reg, b_reg, W_dir, b_dir):
    B, C, H, W = x.shape
    HW = H * W
    Oc = W_cls.shape[0]
    Og = W_reg.shape[0]
    Od = W_dir.shape[0]
    O = Oc + Og + Od
    wt = jnp.concatenate([W_cls, W_reg, W_dir], axis=0).T
    b_all = jnp.concatenate([b_cls, b_reg, b_dir], axis=0).reshape(1, O)
    xv = jnp.transpose(x, (0, 2, 3, 1)).reshape(B, HW, C)
    n_s = pl.cdiv(HW, _TILE_S)

    out = pl.pallas_call(
        _head_kernel,
        grid=(B, n_s),
        in_specs=[
            pl.BlockSpec((1, _TILE_S, C), lambda b, s: (b, s, 0)),
            pl.BlockSpec((C, O), lambda b, s: (0, 0)),
            pl.BlockSpec((1, O), lambda b, s: (0, 0)),
        ],
        out_specs=pl.BlockSpec((1, O, _TILE_S), lambda b, s: (b, 0, s)),
        out_shape=jax.ShapeDtypeStruct((B, O, HW), jnp.float32),
        compiler_params=pltpu.CompilerParams(
            dimension_semantics=("parallel", "parallel"),
        ),
    )(xv, wt, b_all)

    out = out.reshape(B, O, H, W)
    return (out[:, 0:Oc], out[:, Oc:Oc + Og], out[:, Oc + Og:])


# speedup vs baseline: 2.1030x; 1.1032x over previous
import jax
import jax.numpy as jnp
from jax.experimental import pallas as pl
from jax.experimental.pallas import tpu as pltpu

_TILE_S = 2048


def _head_kernel(x_ref, w_ref, b_ref, oc_ref, og_ref, od_ref):
    oc = oc_ref.shape[1]
    og = og_ref.shape[1]
    acc = jnp.dot(x_ref[0], w_ref[:],
                  preferred_element_type=jnp.float32) + b_ref[:]
    acc_t = acc.T
    oc_ref[0] = acc_t[0:oc]
    og_ref[0] = acc_t[oc:oc + og]
    od_ref[0] = acc_t[oc + og:]


@jax.jit
def kernel(x, W_cls, b_cls, W_reg, b_reg, W_dir, b_dir):
    B, C, H, W = x.shape
    HW = H * W
    Oc = W_cls.shape[0]
    Og = W_reg.shape[0]
    Od = W_dir.shape[0]
    O = Oc + Og + Od
    wt = jnp.concatenate([W_cls, W_reg, W_dir], axis=0).T
    b_all = jnp.concatenate([b_cls, b_reg, b_dir], axis=0).reshape(1, O)
    xv = jnp.transpose(x, (0, 2, 3, 1)).reshape(B, HW, C)
    n_s = pl.cdiv(HW, _TILE_S)

    out = pl.pallas_call(
        _head_kernel,
        grid=(B, n_s),
        in_specs=[
            pl.BlockSpec((1, _TILE_S, C), lambda b, s: (b, s, 0)),
            pl.BlockSpec((C, O), lambda b, s: (0, 0)),
            pl.BlockSpec((1, O), lambda b, s: (0, 0)),
        ],
        out_specs=(
            pl.BlockSpec((1, Oc, _TILE_S), lambda b, s: (b, 0, s)),
            pl.BlockSpec((1, Og, _TILE_S), lambda b, s: (b, 0, s)),
            pl.BlockSpec((1, Od, _TILE_S), lambda b, s: (b, 0, s)),
        ),
        out_shape=(
            jax.ShapeDtypeStruct((B, Oc, HW), jnp.float32),
            jax.ShapeDtypeStruct((B, Og, HW), jnp.float32),
            jax.ShapeDtypeStruct((B, Od, HW), jnp.float32),
        ),
        compiler_params=pltpu.CompilerParams(
            dimension_semantics=("parallel", "parallel"),
        ),
    )(xv, wt, b_all)

    out_cls, out_reg, out_dir = out
    return (out_cls.reshape(B, Oc, H, W),
            out_reg.reshape(B, Og, H, W),
            out_dir.reshape(B, Od, H, W))


# speedup vs baseline: 2.2033x; 1.0477x over previous
import jax
import jax.numpy as jnp
from jax.experimental import pallas as pl
from jax.experimental.pallas import tpu as pltpu

_TILE_H = 8


def _head_kernel(x_ref, w_ref, b_ref, oc_ref, og_ref, od_ref):
    oc = oc_ref.shape[2]
    og = og_ref.shape[2]
    b = b_ref[:]
    for h in range(x_ref.shape[1]):
        acc = jnp.dot(x_ref[0, h], w_ref[:],
                      preferred_element_type=jnp.float32) + b
        acc_t = acc.T
        oc_ref[0, h] = acc_t[0:oc]
        og_ref[0, h] = acc_t[oc:oc + og]
        od_ref[0, h] = acc_t[oc + og:]


@jax.jit
def kernel(x, W_cls, b_cls, W_reg, b_reg, W_dir, b_dir):
    B, C, H, W = x.shape
    Oc = W_cls.shape[0]
    Og = W_reg.shape[0]
    Od = W_dir.shape[0]
    O = Oc + Og + Od
    wt = jnp.concatenate([W_cls, W_reg, W_dir], axis=0).T
    b_all = jnp.concatenate([b_cls, b_reg, b_dir], axis=0).reshape(1, O)
    xv = jnp.transpose(x, (0, 2, 3, 1))
    n_h = pl.cdiv(H, _TILE_H)

    def o_spec(o):
        return pl.BlockSpec((1, _TILE_H, o, W), lambda b, h: (b, h, 0, 0))

    out_cls, out_reg, out_dir = pl.pallas_call(
        _head_kernel,
        grid=(B, n_h),
        in_specs=[
            pl.BlockSpec((1, _TILE_H, W, C), lambda b, h: (b, h, 0, 0)),
            pl.BlockSpec((C, O), lambda b, h: (0, 0)),
            pl.BlockSpec((1, O), lambda b, h: (0, 0)),
        ],
        out_specs=(o_spec(Oc), o_spec(Og), o_spec(Od)),
        out_shape=(
            jax.ShapeDtypeStruct((B, H, Oc, W), jnp.float32),
            jax.ShapeDtypeStruct((B, H, Og, W), jnp.float32),
            jax.ShapeDtypeStruct((B, H, Od, W), jnp.float32),
        ),
        compiler_params=pltpu.CompilerParams(
            dimension_semantics=("parallel", "parallel"),
        ),
    )(xv, wt, b_all)

    return (jnp.transpose(out_cls, (0, 2, 1, 3)),
            jnp.transpose(out_reg, (0, 2, 1, 3)),
            jnp.transpose(out_dir, (0, 2, 1, 3)))
